# baseline (speedup 1.0000x reference)
import functools

import jax
import jax.numpy as jnp
from jax import lax
from jax.experimental import pallas as pl
from jax.experimental.pallas import tpu as pltpu
from jax.experimental.pallas import tpu_sc as plsc

_LATENT = 3
_D = _LATENT * 64 * 64
_B = 4096
_NC = 2
_NS = 16
_NW = _NC * _NS
_K = 2
_BK = _B // _K
_BPW = _BK // _NW
_CH = 4
_NCHUNK = _BPW // _CH


def _build_gather():
    mesh = plsc.VectorSubcoreMesh(core_axis_name="c", subcore_axis_name="s")

    @functools.partial(
        pl.kernel,
        mesh=mesh,
        out_type=jax.ShapeDtypeStruct((_BK, _D), jnp.float32),
        scratch_types=[
            pltpu.VMEM((_NCHUNK, _CH), jnp.int32),
            pltpu.VMEM((2, _CH, _D), jnp.float32),
            pltpu.SemaphoreType.DMA,
            pltpu.SemaphoreType.DMA,
        ],
    )
    def gather(idx_hbm, table_hbm, out_hbm, idx_v, rows_v, gsem, ssem):
        wid = lax.axis_index("s") * _NC + lax.axis_index("c")
        base = wid * _BPW
        pltpu.sync_copy(idx_hbm.at[wid], idx_v)
        pltpu.async_copy(table_hbm.at[idx_v.at[0]], rows_v.at[0], gsem)

        def body(g, carry):
            buf = lax.rem(g, 2)
            pltpu.make_async_copy(
                table_hbm.at[idx_v.at[g]], rows_v.at[buf], gsem
            ).wait()

            pltpu.async_copy(
                rows_v.at[buf], out_hbm.at[pl.ds(base + g * _CH, _CH)], ssem
            )

            @pl.when(g >= 1)
            def _drain_prev():
                pltpu.make_async_copy(
                    rows_v.at[1 - buf], out_hbm.at[pl.ds(base, _CH)], ssem
                ).wait()

            @pl.when(g + 1 < _NCHUNK)
            def _start_next():
                pltpu.async_copy(
                    table_hbm.at[idx_v.at[g + 1]], rows_v.at[1 - buf], gsem
                )

            return carry

        lax.fori_loop(0, _NCHUNK, body, 0)
        pltpu.make_async_copy(
            rows_v.at[1], out_hbm.at[pl.ds(base, _CH)], ssem
        ).wait()

    return gather


_GATHER = _build_gather()


def kernel(x, table):
    idx = x.astype(jnp.int32).reshape(_K, _NW, _NCHUNK, _CH)
    outs = [
        _GATHER(idx[k], table).reshape(_BK, _LATENT, 64, 64)
        for k in range(_K)
    ]
    return jnp.stack(outs, axis=0).reshape(_B, _LATENT, 64, 64)


# speedup vs baseline: 1.7577x; 1.7577x over previous
import functools

import jax
import jax.numpy as jnp
from jax import lax
from jax.experimental import pallas as pl
from jax.experimental.pallas import tpu as pltpu
from jax.experimental.pallas import tpu_sc as plsc

_LATENT = 3
_D = _LATENT * 64 * 64
_B = 4096
_NC = 2
_NS = 16
_NW = _NC * _NS
_BPW = _B // _NW
_CH = 2
_NBUF = 4
_NCHUNK = _BPW // _CH


def _build_gather():
    mesh = plsc.VectorSubcoreMesh(core_axis_name="c", subcore_axis_name="s")

    @functools.partial(
        pl.kernel,
        mesh=mesh,
        out_type=jax.ShapeDtypeStruct((_B, _D), jnp.float32),
        scratch_types=[
            pltpu.VMEM((_NCHUNK, _CH), jnp.int32),
            pltpu.VMEM((_NBUF, _CH, _D), jnp.float32),
            pltpu.SemaphoreType.DMA,
            pltpu.SemaphoreType.DMA,
        ],
    )
    def gather(idx_hbm, table_hbm, out_hbm, idx_v, rows_v, gsem, ssem):
        wid = lax.axis_index("s") * _NC + lax.axis_index("c")
        base = wid * _BPW
        pltpu.sync_copy(idx_hbm.at[wid], idx_v)
        for b in range(_NBUF - 1):
            pltpu.async_copy(table_hbm.at[idx_v.at[b]], rows_v.at[b], gsem)

        def body(g, carry):
            buf = lax.rem(g, _NBUF)
            pltpu.make_async_copy(
                table_hbm.at[idx_v.at[g]], rows_v.at[buf], gsem
            ).wait()

            pltpu.async_copy(
                rows_v.at[buf], out_hbm.at[pl.ds(base + g * _CH, _CH)], ssem
            )

            @pl.when(g >= 1)
            def _drain_prev():
                pltpu.make_async_copy(
                    rows_v.at[buf], out_hbm.at[pl.ds(base, _CH)], ssem
                ).wait()

            @pl.when(g + _NBUF - 1 < _NCHUNK)
            def _start_next():
                nb = lax.rem(g + _NBUF - 1, _NBUF)
                pltpu.async_copy(
                    table_hbm.at[idx_v.at[g + _NBUF - 1]], rows_v.at[nb], gsem
                )

            return carry

        lax.fori_loop(0, _NCHUNK, body, 0)
        pltpu.make_async_copy(
            rows_v.at[0], out_hbm.at[pl.ds(base, _CH)], ssem
        ).wait()

    return gather


_GATHER = _build_gather()


def kernel(x, table):
    idx = x.astype(jnp.int32).reshape(_NW, _NCHUNK, _CH)
    out = _GATHER(idx, table)
    return out.reshape(-1, _LATENT, 64, 64)
